# baseline (speedup 1.0000x reference)
import functools

import jax
import jax.numpy as jnp
from jax import lax
from jax.experimental import pallas as pl
from jax.experimental.pallas import tpu as pltpu
from jax.experimental.pallas import tpu_sc as plsc

NC = 2
NS = 16
L = 16
NW = NC * NS

CH = 128


def _sc_scatter(n_nodes, d, ep, acc_rows):
    e_per_tile = ep // NW
    n_ch = e_per_tile // CH
    assert n_ch % 2 == 0
    rows_per_tile = acc_rows // NS
    n_zero = rows_per_tile // CH

    mesh = plsc.VectorSubcoreMesh(core_axis_name="c", subcore_axis_name="s")

    @functools.partial(
        pl.kernel,
        mesh=mesh,
        out_type=jax.ShapeDtypeStruct((NC, acc_rows, d), jnp.float32),
        scratch_types=[
            pltpu.VMEM((2, CH), jnp.int32),
            pltpu.VMEM((2, CH), jnp.int32),
            pltpu.VMEM((CH, d), jnp.float32),
            pltpu.VMEM((CH, d), jnp.float32),
            pltpu.VMEM_SHARED((acc_rows, d), jnp.float32),
            pltpu.SemaphoreType.DMA,
            pltpu.SemaphoreType.DMA,
            pltpu.SemaphoreType.DMA,
            pltpu.SemaphoreType.DMA,
        ],
    )
    def k(feat_hbm, edge_hbm, out_hbm, idx0, idx1, rows0, rows1, acc_sh,
          sem_g0, sem_g1, sem_s0, sem_s1):
        cid = lax.axis_index("c")
        sid = lax.axis_index("s")
        wid = sid * NC + cid

        def zero_body(i, _):
            rows0[i // (d // L), pl.ds((i % (d // L)) * L, L)] = jnp.zeros(
                (L,), jnp.float32)
            return _
        lax.fori_loop(0, CH * (d // L), zero_body, None)
        acc_base = sid * rows_per_tile
        for j in range(n_zero):
            pltpu.sync_copy(rows0, acc_sh.at[pl.ds(acc_base + j * CH, CH)])
        plsc.subcore_barrier()

        ebase = wid * e_per_tile

        def fetch_idx(idx, c):
            pltpu.sync_copy(edge_hbm.at[:, pl.ds(ebase + c * CH, CH)], idx)

        def start_gather(idx, rows, sem):
            pltpu.async_copy(feat_hbm.at[idx.at[0]], rows, sem)

        def wait_gather(idx, rows, sem):
            pltpu.make_async_copy(feat_hbm.at[idx.at[0]], rows, sem).wait()

        def start_scatter(idx, rows, sem):
            pltpu.async_copy(rows, acc_sh.at[idx.at[1]], sem, add=True)

        def wait_scatter(idx, rows, sem):
            pltpu.make_async_copy(rows, acc_sh.at[idx.at[1]], sem).wait()

        fetch_idx(idx0, 0)
        start_gather(idx0, rows0, sem_g0)

        def edge_body(g, _):
            @pl.when(g > 0)
            def _wait_prev_odd():
                wait_scatter(idx1, rows1, sem_s1)
            fetch_idx(idx1, 2 * g + 1)
            wait_gather(idx0, rows0, sem_g0)
            start_scatter(idx0, rows0, sem_s0)
            start_gather(idx1, rows1, sem_g1)
            wait_scatter(idx0, rows0, sem_s0)
            fetch_idx(idx0, jnp.minimum(2 * g + 2, n_ch - 1))
            start_gather(idx0, rows0, sem_g0)
            wait_gather(idx1, rows1, sem_g1)
            start_scatter(idx1, rows1, sem_s1)
            return _
        lax.fori_loop(0, n_ch // 2, edge_body, None)
        wait_gather(idx0, rows0, sem_g0)
        wait_scatter(idx1, rows1, sem_s1)
        plsc.subcore_barrier()

        pltpu.sync_copy(
            acc_sh.at[pl.ds(acc_base, rows_per_tile)],
            out_hbm.at[cid, pl.ds(acc_base, rows_per_tile)],
        )

    return k


def _combine_body(p0_ref, p1_ref, w_ref, o_ref):
    o_ref[...] = (p0_ref[0] + p1_ref[0]) * w_ref[...]


def kernel(features, edge_index, W):
    n_nodes, d = features.shape
    e = edge_index.shape[1]

    ep = -(-e // (2 * NW * CH)) * (2 * NW * CH)
    ei = edge_index
    if ep != e:
        pad = ep - e
        ei = jnp.concatenate(
            [ei, jnp.stack([jnp.zeros((pad,), jnp.int32),
                            jnp.full((pad,), n_nodes, jnp.int32)])], axis=1)

    acc_rows = -(-(n_nodes + 1) // (NS * CH)) * (NS * CH)
    partial = _sc_scatter(n_nodes, d, ep, acc_rows)(features, ei)

    blk = 1000
    grid = n_nodes // blk
    out = pl.pallas_call(
        _combine_body,
        grid=(grid,),
        in_specs=[
            pl.BlockSpec((1, blk, d), lambda i: (0, i, 0)),
            pl.BlockSpec((1, blk, d), lambda i: (1, i, 0)),
            pl.BlockSpec((1, d), lambda i: (0, 0)),
        ],
        out_specs=pl.BlockSpec((blk, d), lambda i: (i, 0)),
        out_shape=jax.ShapeDtypeStruct((n_nodes, d), jnp.float32),
    )(partial, partial, W.reshape(1, d))
    return out
